# baseline (speedup 1.0000x reference)
import functools

import jax
import jax.numpy as jnp
from jax import lax
from jax.experimental import pallas as pl
from jax.experimental.pallas import tpu as pltpu
from jax.experimental.pallas import tpu_sc as plsc

_D_MODEL = 2112
_MAX_LENGTH = 2048
_EMBED_DIM = _D_MODEL - _MAX_LENGTH
_LANES = 128
_NSLOT = 8


_GPW = 245
_CAP = 352


def _make_sc_gather(B, D):
    info = plsc.get_sparse_core_info()
    NC, NS = info.num_cores, info.num_subcores
    NW = NC * NS
    n_vregs = B // 16
    mesh = plsc.VectorSubcoreMesh(core_axis_name="c", subcore_axis_name="s")

    @functools.partial(
        pl.kernel,
        mesh=mesh,
        out_type=jax.ShapeDtypeStruct((B, D), jnp.float32),
        scratch_types=[
            pltpu.VMEM((B,), jnp.int32),
            pltpu.VMEM((_CAP,), jnp.int32),
            pltpu.VMEM((_CAP,), jnp.int32),
            pltpu.VMEM((256,), jnp.int32),
            pltpu.VMEM((272,), jnp.int32),
            pltpu.VMEM((_NSLOT, D, _LANES), jnp.float32),
            pltpu.VMEM((_CAP, D), jnp.float32),
            pltpu.SemaphoreType.DMA((_NSLOT,)),
            pltpu.SemaphoreType.DMA,
        ],
        compiler_params=pltpu.CompilerParams(needs_layout_passes=False),
    )
    def gather_kernel(wt_hbm, idx_hbm, out_hbm, idx_all, mine_x, mine_t,
                      flags, glist, grp_v, crow_v, sem, osem):
        wid = lax.axis_index("s") * NC + lax.axis_index("c")
        glo = wid * _GPW
        lanes16 = lax.iota(jnp.int32, 16)
        pltpu.sync_copy(idx_hbm, idx_all)

        for v in range(_CAP // 16):
            mine_x[pl.ds(16 * v, 16)] = jnp.full((16,), -1, jnp.int32)
        for v in range(256 // 16):
            flags[pl.ds(16 * v, 16)] = jnp.zeros((16,), jnp.int32)
        for v in range(272 // 16):
            glist[pl.ds(16 * v, 16)] = jnp.full((16,), -3, jnp.int32)

        def scan_body(c, off):
            v = idx_all[pl.ds(c * 16, 16)]
            g = v >> 7
            m = jnp.logical_and(g >= glo, g < glo + _GPW)
            offc = jnp.minimum(off, _CAP - 16)
            plsc.store_compressed(mine_x.at[pl.ds(offc, 16)], v, mask=m)
            plsc.store_compressed(mine_t.at[pl.ds(offc, 16)],
                                  lanes16 + c * 16, mask=m)
            return off + plsc.all_reduce_population_count(m)[0]

        ntok = pl.loop(0, n_vregs, init_carry=jnp.int32(0))(scan_body)
        ntok = jnp.minimum(ntok, _CAP)

        ones = jnp.ones((16,), jnp.int32)
        for v in range(_CAP // 16):
            rel = (mine_x[pl.ds(16 * v, 16)] >> 7) - glo
            plsc.store_scatter(flags, [rel], ones, mask=rel >= 0)
        ng = jnp.int32(0)
        for c in range(256 // 16):
            fv = flags[pl.ds(16 * c, 16)]
            m = fv > 0
            plsc.store_compressed(glist.at[pl.ds(ng, 16)],
                                  lanes16 + 16 * c, mask=m)
            ng = ng + plsc.all_reduce_population_count(m)[0]
        nbatch = (ng + 7) >> 3

        def fire(rel_gid, slot):
            geff = jnp.maximum(rel_gid, 0)
            col0 = pl.multiple_of((glo + geff) * _LANES, _LANES)
            pltpu.async_copy(wt_hbm.at[:, pl.ds(col0, _LANES)],
                             grp_v.at[slot], sem.at[slot])

        def drain(slot):
            pltpu.make_async_copy(wt_hbm.at[:, pl.ds(0, _LANES)],
                                  grp_v.at[slot], sem.at[slot]).wait()

        def extract_group(rel_gid, slot):
            def per_vreg(v):
                xm = mine_x[pl.ds(v * 16, 16)]
                m0 = ((xm >> 7) - glo) == rel_gid

                def ext_one(m):
                    l = plsc.all_reduce_ffs(m)[0]
                    lsp = jnp.broadcast_to(l, (16,))
                    xv = xm.at[lsp].get(mode="promise_in_bounds")[0]
                    row = v * 16 + l
                    lane = jnp.broadcast_to(xv & (_LANES - 1), (16,))
                    for k in range(D // 16):
                        val = plsc.load_gather(grp_v.at[slot],
                                               [lanes16 + 16 * k, lane])
                        crow_v[row, pl.ds(16 * k, 16)] = val
                    return jnp.logical_and(m, lanes16 != l)

                lax.while_loop(jnp.any, ext_one, m0)

            pl.loop(0, _CAP // 16)(per_vreg)

        def batch(c, _):
            wcur = glist[pl.ds(c * 8, 16)]
            poff = jnp.maximum((c - 1) * 8, 0)
            wprev = glist[pl.ds(poff, 16)]
            for j in range(8):
                @pl.when(c > 0)
                def _():
                    drain(j)
                    extract_group(wprev[j], j)
                fire(wcur[j], j)
            return 0

        pl.loop(0, nbatch, init_carry=jnp.int32(0))(batch)

        @pl.when(nbatch > 0)
        def _():
            wlast = glist[pl.ds(jnp.maximum(nbatch - 1, 0) * 8, 16)]
            for j in range(8):
                drain(j)
                extract_group(wlast[j], j)

        def wout(r, _):
            base16 = (r >> 4) << 4
            win = mine_t[pl.ds(base16, 16)]
            tv = win.at[jnp.broadcast_to(r & 15, (16,))].get(
                mode="promise_in_bounds")[0]
            pltpu.async_copy(crow_v.at[pl.ds(r, 1)],
                             out_hbm.at[pl.ds(tv, 1)], osem)

            @pl.when(r >= 32)
            def _():
                pltpu.make_async_copy(crow_v.at[pl.ds(0, 1)],
                                      out_hbm.at[pl.ds(0, 1)], osem).wait()
            return 0

        pl.loop(0, ntok, init_carry=jnp.int32(0))(wout)

        def wdrain(r, _):
            pltpu.make_async_copy(crow_v.at[pl.ds(0, 1)],
                                  out_hbm.at[pl.ds(0, 1)], osem).wait()
            return 0

        pl.loop(0, jnp.minimum(ntok, 32), init_carry=jnp.int32(0))(wdrain)

    return gather_kernel


_CB = 528


def _pe_body(pos_ref, out_ref):
    j = pl.program_id(1)
    chan = lax.broadcasted_iota(jnp.int32, (_CB, _MAX_LENGTH), 0)
    target = chan + (j * _CB - _EMBED_DIM)
    out_ref[0] = (target == pos_ref[0]).astype(jnp.float32)


def _pe_write(pos3, batch, seq):
    grid = (batch, _D_MODEL // _CB)
    return pl.pallas_call(
        _pe_body,
        grid=grid,
        in_specs=[pl.BlockSpec((1, 1, seq), lambda b, j: (b, 0, 0))],
        out_specs=pl.BlockSpec((1, _CB, seq), lambda b, j: (b, j, 0)),
        out_shape=jax.ShapeDtypeStruct((batch, _D_MODEL, seq), jnp.float32),
    )(pos3)


def _tok_body(pe_ref, tok_ref, out_ref):
    del pe_ref
    row = lax.broadcasted_iota(jnp.int32, (_EMBED_DIM, _EMBED_DIM), 0)
    col = lax.broadcasted_iota(jnp.int32, (_EMBED_DIM, _EMBED_DIM), 1)
    eye = (row == col).astype(jnp.float32)
    out_ref[0] = lax.dot_general(eye, tok_ref[...],
                                 (((1,), (1,)), ((), ())),
                                 preferred_element_type=jnp.float32)


def _tok_write(pe, tok, batch, seq):
    return pl.pallas_call(
        _tok_body,
        grid=(batch,),
        in_specs=[
            pl.BlockSpec(memory_space=pl.ANY),
            pl.BlockSpec((seq, _EMBED_DIM), lambda b: (b, 0)),
        ],
        out_specs=pl.BlockSpec((1, _EMBED_DIM, seq), lambda b: (b, 0, 0)),
        out_shape=jax.ShapeDtypeStruct((batch, _D_MODEL, seq), jnp.float32),
        input_output_aliases={0: 0},
    )(pe, tok)


def kernel(x, pos, token_embed_weight):
    batch, seq = x.shape
    B = batch * seq
    x_flat = x.reshape(B).astype(jnp.int32)
    pos3 = pos.reshape(batch, 1, seq).astype(jnp.int32)
    w_t = token_embed_weight.T
    tok = _make_sc_gather(B, _EMBED_DIM)(w_t, x_flat)
    pe = _pe_write(pos3, batch, seq)
    out_t = _tok_write(pe, tok, batch, seq)
    return jnp.swapaxes(out_t, 1, 2)


# speedup vs baseline: 1.1325x; 1.1325x over previous
import functools

import jax
import jax.numpy as jnp
from jax import lax
from jax.experimental import pallas as pl
from jax.experimental.pallas import tpu as pltpu
from jax.experimental.pallas import tpu_sc as plsc

_D_MODEL = 2112
_MAX_LENGTH = 2048
_EMBED_DIM = _D_MODEL - _MAX_LENGTH
_LANES = 128
_NSLOT = 8


_GPW = 245
_CAP = 352


def _make_sc_gather(B, D):
    info = plsc.get_sparse_core_info()
    NC, NS = info.num_cores, info.num_subcores
    NW = NC * NS
    n_vregs = B // 16
    mesh = plsc.VectorSubcoreMesh(core_axis_name="c", subcore_axis_name="s")

    @functools.partial(
        pl.kernel,
        mesh=mesh,
        out_type=jax.ShapeDtypeStruct((B, D), jnp.float32),
        scratch_types=[
            pltpu.VMEM((B,), jnp.int32),
            pltpu.VMEM((_CAP,), jnp.int32),
            pltpu.VMEM((_CAP,), jnp.int32),
            pltpu.VMEM((256,), jnp.int32),
            pltpu.VMEM((256,), jnp.int32),
            pltpu.VMEM((_CAP,), jnp.int32),
            pltpu.VMEM((272,), jnp.int32),
            pltpu.VMEM((_NSLOT, D, _LANES), jnp.float32),
            pltpu.VMEM((_CAP, D), jnp.float32),
            pltpu.SemaphoreType.DMA((_NSLOT,)),
            pltpu.SemaphoreType.DMA,
        ],
        compiler_params=pltpu.CompilerParams(needs_layout_passes=False),
    )
    def gather_kernel(wt_hbm, idx_hbm, out_hbm, idx_all, mine_x, mine_t,
                      flags, csum, mine_o, glist, grp_v, crow_v, sem, osem):
        wid = lax.axis_index("s") * NC + lax.axis_index("c")
        glo = wid * _GPW
        lanes16 = lax.iota(jnp.int32, 16)
        pltpu.sync_copy(idx_hbm, idx_all)

        for v in range(_CAP // 16):
            mine_x[pl.ds(16 * v, 16)] = jnp.full((16,), -1, jnp.int32)
        for v in range(256 // 16):
            flags[pl.ds(16 * v, 16)] = jnp.zeros((16,), jnp.int32)
        for v in range(272 // 16):
            glist[pl.ds(16 * v, 16)] = jnp.full((16,), -3, jnp.int32)

        def scan_body(c, off):
            v = idx_all[pl.ds(c * 16, 16)]
            g = v >> 7
            m = jnp.logical_and(g >= glo, g < glo + _GPW)
            offc = jnp.minimum(off, _CAP - 16)
            plsc.store_compressed(mine_x.at[pl.ds(offc, 16)], v, mask=m)
            plsc.store_compressed(mine_t.at[pl.ds(offc, 16)],
                                  lanes16 + c * 16, mask=m)
            return off + plsc.all_reduce_population_count(m)[0]

        ntok = pl.loop(0, n_vregs, init_carry=jnp.int32(0))(scan_body)
        ntok = jnp.minimum(ntok, _CAP)

        ones = jnp.ones((16,), jnp.int32)
        for v in range(_CAP // 16):
            rel = (mine_x[pl.ds(16 * v, 16)] >> 7) - glo
            plsc.store_scatter(flags, [rel], ones, mask=rel >= 0)
        ng = jnp.int32(0)
        for c in range(256 // 16):
            fv = flags[pl.ds(16 * c, 16)]
            m = fv > 0
            plsc.store_compressed(glist.at[pl.ds(ng, 16)],
                                  lanes16 + 16 * c, mask=m)
            csum[pl.ds(16 * c, 16)] = plsc.cumsum(fv) + ng
            ng = ng + plsc.all_reduce_population_count(m)[0]
        nbatch = (ng + 7) >> 3

        for v in range(_CAP // 16):
            rel = (mine_x[pl.ds(16 * v, 16)] >> 7) - glo
            o = plsc.load_gather(csum, [jnp.maximum(rel, 0)]) - 1
            mine_o[pl.ds(16 * v, 16)] = jnp.where(rel >= 0, o, -1)

        def fire(rel_gid, slot):
            geff = jnp.maximum(rel_gid, 0)
            col0 = pl.multiple_of((glo + geff) * _LANES, _LANES)
            pltpu.async_copy(wt_hbm.at[:, pl.ds(col0, _LANES)],
                             grp_v.at[slot], sem.at[slot])

        def drain(slot):
            pltpu.make_async_copy(wt_hbm.at[:, pl.ds(0, _LANES)],
                                  grp_v.at[slot], sem.at[slot]).wait()

        def extract_batch(c):
            def per_vreg(v):
                ov = mine_o[pl.ds(v * 16, 16)]
                m0 = (ov >> 3) == c
                xm = mine_x[pl.ds(v * 16, 16)]

                def ext_one(m):
                    l = plsc.all_reduce_ffs(m)[0]
                    lsp = jnp.broadcast_to(l, (16,))
                    xv = xm.at[lsp].get(mode="promise_in_bounds")[0]
                    slot = ov.at[lsp].get(mode="promise_in_bounds")[0] & 7
                    row = v * 16 + l
                    lane = jnp.broadcast_to(xv & (_LANES - 1), (16,))
                    ssp = jnp.broadcast_to(slot, (16,))
                    for k in range(D // 16):
                        val = plsc.load_gather(
                            grp_v, [ssp, lanes16 + 16 * k, lane])
                        crow_v[row, pl.ds(16 * k, 16)] = val
                    return jnp.logical_and(m, lanes16 != l)

                lax.while_loop(jnp.any, ext_one, m0)

            pl.loop(0, _CAP // 16)(per_vreg)

        @pl.when(nbatch > 0)
        def _():
            w0 = glist[pl.ds(0, 16)]
            for j in range(8):
                fire(w0[j], j)

        def batch(c, _):
            for j in range(8):
                drain(j)
            extract_batch(c)

            @pl.when(c + 1 < nbatch)
            def _():
                wnext = glist[pl.ds((c + 1) * 8, 16)]
                for j in range(8):
                    fire(wnext[j], j)
            return 0

        pl.loop(0, nbatch, init_carry=jnp.int32(0))(batch)

        def wout(r, _):
            base16 = (r >> 4) << 4
            win = mine_t[pl.ds(base16, 16)]
            tv = win.at[jnp.broadcast_to(r & 15, (16,))].get(
                mode="promise_in_bounds")[0]
            pltpu.async_copy(crow_v.at[pl.ds(r, 1)],
                             out_hbm.at[pl.ds(tv, 1)], osem)

            @pl.when(r >= 32)
            def _():
                pltpu.make_async_copy(crow_v.at[pl.ds(0, 1)],
                                      out_hbm.at[pl.ds(0, 1)], osem).wait()
            return 0

        pl.loop(0, ntok, init_carry=jnp.int32(0))(wout)

        def wdrain(r, _):
            pltpu.make_async_copy(crow_v.at[pl.ds(0, 1)],
                                  out_hbm.at[pl.ds(0, 1)], osem).wait()
            return 0

        pl.loop(0, jnp.minimum(ntok, 32), init_carry=jnp.int32(0))(wdrain)

    return gather_kernel


_CB = 528


def _pe_body(pos_ref, out_ref):
    j = pl.program_id(1)
    chan = lax.broadcasted_iota(jnp.int32, (_CB, _MAX_LENGTH), 0)
    target = chan + (j * _CB - _EMBED_DIM)
    out_ref[0] = (target == pos_ref[0]).astype(jnp.float32)


def _pe_write(pos3, batch, seq):
    grid = (batch, _D_MODEL // _CB)
    return pl.pallas_call(
        _pe_body,
        grid=grid,
        in_specs=[pl.BlockSpec((1, 1, seq), lambda b, j: (b, 0, 0))],
        out_specs=pl.BlockSpec((1, _CB, seq), lambda b, j: (b, j, 0)),
        out_shape=jax.ShapeDtypeStruct((batch, _D_MODEL, seq), jnp.float32),
    )(pos3)


def _tok_body(pe_ref, tok_ref, out_ref):
    del pe_ref
    row = lax.broadcasted_iota(jnp.int32, (_EMBED_DIM, _EMBED_DIM), 0)
    col = lax.broadcasted_iota(jnp.int32, (_EMBED_DIM, _EMBED_DIM), 1)
    eye = (row == col).astype(jnp.float32)
    out_ref[0] = lax.dot_general(eye, tok_ref[...],
                                 (((1,), (1,)), ((), ())),
                                 preferred_element_type=jnp.float32)


def _tok_write(pe, tok, batch, seq):
    return pl.pallas_call(
        _tok_body,
        grid=(batch,),
        in_specs=[
            pl.BlockSpec(memory_space=pl.ANY),
            pl.BlockSpec((seq, _EMBED_DIM), lambda b: (b, 0)),
        ],
        out_specs=pl.BlockSpec((1, _EMBED_DIM, seq), lambda b: (b, 0, 0)),
        out_shape=jax.ShapeDtypeStruct((batch, _D_MODEL, seq), jnp.float32),
        input_output_aliases={0: 0},
    )(pe, tok)


def kernel(x, pos, token_embed_weight):
    batch, seq = x.shape
    B = batch * seq
    x_flat = x.reshape(B).astype(jnp.int32)
    pos3 = pos.reshape(batch, 1, seq).astype(jnp.int32)
    w_t = token_embed_weight.T
    tok = _make_sc_gather(B, _EMBED_DIM)(w_t, x_flat)
    pe = _pe_write(pos3, batch, seq)
    out_t = _tok_write(pe, tok, batch, seq)
    return jnp.swapaxes(out_t, 1, 2)


# speedup vs baseline: 1.3836x; 1.2216x over previous
import functools

import jax
import jax.numpy as jnp
from jax import lax
from jax.experimental import pallas as pl
from jax.experimental.pallas import tpu as pltpu
from jax.experimental.pallas import tpu_sc as plsc

_D_MODEL = 2112
_MAX_LENGTH = 2048
_EMBED_DIM = _D_MODEL - _MAX_LENGTH
_LANES = 128
_NSLOT = 8


_GPW = 245
_CAP = 352


def _make_sc_gather(B, D):
    info = plsc.get_sparse_core_info()
    NC, NS = info.num_cores, info.num_subcores
    NW = NC * NS
    n_vregs = B // 16
    mesh = plsc.VectorSubcoreMesh(core_axis_name="c", subcore_axis_name="s")

    @functools.partial(
        pl.kernel,
        mesh=mesh,
        out_type=jax.ShapeDtypeStruct((B, D), jnp.float32),
        scratch_types=[
            pltpu.VMEM((B,), jnp.int32),
            pltpu.VMEM((_CAP,), jnp.int32),
            pltpu.VMEM((_CAP,), jnp.int32),
            pltpu.VMEM((256,), jnp.int32),
            pltpu.VMEM((256,), jnp.int32),
            pltpu.VMEM((_CAP,), jnp.int32),
            pltpu.VMEM((272,), jnp.int32),
            pltpu.VMEM((_NSLOT, D, _LANES), jnp.float32),
            pltpu.VMEM((_CAP, D), jnp.float32),
            pltpu.SemaphoreType.DMA((_NSLOT,)),
            pltpu.SemaphoreType.DMA,
        ],
        compiler_params=pltpu.CompilerParams(needs_layout_passes=False),
    )
    def gather_kernel(wt_hbm, idx_hbm, out_hbm, idx_all, mine_x, mine_t,
                      flags, csum, mine_o, glist, grp_v, crow_v, sem, osem):
        wid = lax.axis_index("s") * NC + lax.axis_index("c")
        glo = wid * _GPW
        lanes16 = lax.iota(jnp.int32, 16)
        pltpu.sync_copy(idx_hbm, idx_all)

        for v in range(_CAP // 16):
            mine_x[pl.ds(16 * v, 16)] = jnp.full((16,), -1, jnp.int32)
        for v in range(256 // 16):
            flags[pl.ds(16 * v, 16)] = jnp.zeros((16,), jnp.int32)
        for v in range(272 // 16):
            glist[pl.ds(16 * v, 16)] = jnp.full((16,), -3, jnp.int32)

        def scan_body(c, off):
            v = idx_all[pl.ds(c * 16, 16)]
            g = v >> 7
            m = jnp.logical_and(g >= glo, g < glo + _GPW)
            offc = jnp.minimum(off, _CAP - 16)
            plsc.store_compressed(mine_x.at[pl.ds(offc, 16)], v, mask=m)
            plsc.store_compressed(mine_t.at[pl.ds(offc, 16)],
                                  lanes16 + c * 16, mask=m)
            return off + plsc.all_reduce_population_count(m)[0]

        ntok = pl.loop(0, n_vregs, init_carry=jnp.int32(0))(scan_body)
        ntok = jnp.minimum(ntok, _CAP)

        ones = jnp.ones((16,), jnp.int32)
        for v in range(_CAP // 16):
            rel = (mine_x[pl.ds(16 * v, 16)] >> 7) - glo
            plsc.store_scatter(flags, [rel], ones, mask=rel >= 0)
        ng = jnp.int32(0)
        for c in range(256 // 16):
            fv = flags[pl.ds(16 * c, 16)]
            m = fv > 0
            plsc.store_compressed(glist.at[pl.ds(ng, 16)],
                                  lanes16 + 16 * c, mask=m)
            csum[pl.ds(16 * c, 16)] = plsc.cumsum(fv) + ng
            ng = ng + plsc.all_reduce_population_count(m)[0]
        nbatch = (ng + 3) >> 2

        for v in range(_CAP // 16):
            rel = (mine_x[pl.ds(16 * v, 16)] >> 7) - glo
            o = plsc.load_gather(csum, [jnp.maximum(rel, 0)]) - 1
            mine_o[pl.ds(16 * v, 16)] = jnp.where(rel >= 0, o, -1)

        def fire(rel_gid, slot):
            geff = jnp.maximum(rel_gid, 0)
            col0 = pl.multiple_of((glo + geff) * _LANES, _LANES)
            pltpu.async_copy(wt_hbm.at[:, pl.ds(col0, _LANES)],
                             grp_v.at[slot], sem.at[slot])

        def drain(slot):
            pltpu.make_async_copy(wt_hbm.at[:, pl.ds(0, _LANES)],
                                  grp_v.at[slot], sem.at[slot]).wait()

        def extract_batch(c):
            sbase = (c & 1) * 4

            def per_vreg(v):
                ov = mine_o[pl.ds(v * 16, 16)]
                m0 = (ov >> 2) == c
                xm = mine_x[pl.ds(v * 16, 16)]

                def ext_one(m):
                    l = plsc.all_reduce_ffs(m)[0]
                    lsp = jnp.broadcast_to(l, (16,))
                    xv = xm.at[lsp].get(mode="promise_in_bounds")[0]
                    slot = sbase + (
                        ov.at[lsp].get(mode="promise_in_bounds")[0] & 3)
                    row = v * 16 + l
                    lane = jnp.broadcast_to(xv & (_LANES - 1), (16,))
                    ssp = jnp.broadcast_to(slot, (16,))
                    for k in range(D // 16):
                        val = plsc.load_gather(
                            grp_v, [ssp, lanes16 + 16 * k, lane])
                        crow_v[row, pl.ds(16 * k, 16)] = val
                    return jnp.logical_and(m, lanes16 != l)

                lax.while_loop(jnp.any, ext_one, m0)

            pl.loop(0, _CAP // 16)(per_vreg)

        def fire_batch(c):
            w = glist[pl.ds(c * 4, 16)]

            @pl.when((c & 1) == 0)
            def _():
                for j in range(4):
                    fire(w[j], j)

            @pl.when((c & 1) == 1)
            def _():
                for j in range(4):
                    fire(w[j], 4 + j)

        @pl.when(nbatch > 0)
        def _():
            fire_batch(jnp.int32(0))

        @pl.when(nbatch > 1)
        def _():
            fire_batch(jnp.int32(1))

        def batch(c, _):
            @pl.when((c & 1) == 0)
            def _():
                for j in range(4):
                    drain(j)

            @pl.when((c & 1) == 1)
            def _():
                for j in range(4):
                    drain(4 + j)

            extract_batch(c)

            @pl.when(c + 2 < nbatch)
            def _():
                fire_batch(c + 2)
            return 0

        pl.loop(0, nbatch, init_carry=jnp.int32(0))(batch)

        def wout(r, _):
            base16 = (r >> 4) << 4
            win = mine_t[pl.ds(base16, 16)]
            tv = win.at[jnp.broadcast_to(r & 15, (16,))].get(
                mode="promise_in_bounds")[0]
            pltpu.async_copy(crow_v.at[pl.ds(r, 1)],
                             out_hbm.at[pl.ds(tv, 1)], osem)

            @pl.when(r >= 32)
            def _():
                pltpu.make_async_copy(crow_v.at[pl.ds(0, 1)],
                                      out_hbm.at[pl.ds(0, 1)], osem).wait()
            return 0

        pl.loop(0, ntok, init_carry=jnp.int32(0))(wout)

        def wdrain(r, _):
            pltpu.make_async_copy(crow_v.at[pl.ds(0, 1)],
                                  out_hbm.at[pl.ds(0, 1)], osem).wait()
            return 0

        pl.loop(0, jnp.minimum(ntok, 32), init_carry=jnp.int32(0))(wdrain)

    return gather_kernel


_CB = 528


def _pe_body(pos_ref, out_ref):
    j = pl.program_id(1)
    chan = lax.broadcasted_iota(jnp.int32, (_CB, _MAX_LENGTH), 0)
    target = chan + (j * _CB - _EMBED_DIM)
    out_ref[0] = (target == pos_ref[0]).astype(jnp.float32)


def _pe_write(pos3, batch, seq):
    grid = (batch, _D_MODEL // _CB)
    return pl.pallas_call(
        _pe_body,
        grid=grid,
        in_specs=[pl.BlockSpec((1, 1, seq), lambda b, j: (b, 0, 0))],
        out_specs=pl.BlockSpec((1, _CB, seq), lambda b, j: (b, j, 0)),
        out_shape=jax.ShapeDtypeStruct((batch, _D_MODEL, seq), jnp.float32),
    )(pos3)


def _tok_body(pe_ref, tok_ref, out_ref):
    del pe_ref
    row = lax.broadcasted_iota(jnp.int32, (_EMBED_DIM, _EMBED_DIM), 0)
    col = lax.broadcasted_iota(jnp.int32, (_EMBED_DIM, _EMBED_DIM), 1)
    eye = (row == col).astype(jnp.float32)
    out_ref[0] = lax.dot_general(eye, tok_ref[...],
                                 (((1,), (1,)), ((), ())),
                                 preferred_element_type=jnp.float32)


def _tok_write(pe, tok, batch, seq):
    return pl.pallas_call(
        _tok_body,
        grid=(batch,),
        in_specs=[
            pl.BlockSpec(memory_space=pl.ANY),
            pl.BlockSpec((seq, _EMBED_DIM), lambda b: (b, 0)),
        ],
        out_specs=pl.BlockSpec((1, _EMBED_DIM, seq), lambda b: (b, 0, 0)),
        out_shape=jax.ShapeDtypeStruct((batch, _D_MODEL, seq), jnp.float32),
        input_output_aliases={0: 0},
    )(pe, tok)


def kernel(x, pos, token_embed_weight):
    batch, seq = x.shape
    B = batch * seq
    x_flat = x.reshape(B).astype(jnp.int32)
    pos3 = pos.reshape(batch, 1, seq).astype(jnp.int32)
    w_t = token_embed_weight.T
    tok = _make_sc_gather(B, _EMBED_DIM)(w_t, x_flat)
    pe = _pe_write(pos3, batch, seq)
    out_t = _tok_write(pe, tok, batch, seq)
    return jnp.swapaxes(out_t, 1, 2)
